# Initial kernel scaffold; baseline (speedup 1.0000x reference)
import jax
import jax.numpy as jnp
from jax.experimental import pallas as pl


def kernel(x, edge_index, batch, W_emb, b_emb, W1, b1, W2, b2, W3, b3, Wfc1, bfc1, Wp, bp):
    raise NotImplementedError("write your pallas kernel here")


# speedup vs baseline: 11.7922x; 11.7922x over previous
import functools

import jax
import jax.numpy as jnp
from jax import lax
from jax.experimental import pallas as pl
from jax.experimental.pallas import tpu as pltpu
from jax.experimental.pallas import tpu_sc as plsc

N = 10000
E = 320000
DIN = 128
HID = 256
G = 64
NC = 2
NS = 16
F32 = jnp.float32
I32 = jnp.int32

_KA = 40
_NCA = (E // (NC * NS)) // _KA
_NPAD = 10240

_deg_mesh = plsc.VectorSubcoreMesh(core_axis_name="c", subcore_axis_name="s", num_cores=NC, num_subcores=NS)


@functools.partial(
    pl.kernel,
    out_type=jax.ShapeDtypeStruct((NC, _NPAD, 1), F32),
    mesh=_deg_mesh,
    scratch_types=[
        pltpu.VMEM((_NCA, _KA), I32),
        pltpu.VMEM((_KA, 1), F32),
        pltpu.VMEM_SHARED((_NPAD, 1), F32),
    ],
)
def _deg_kernel(dst_r, ones_r, z1_r, out_r, dstv, onesv, acc):
    c = lax.axis_index("c")
    s = lax.axis_index("s")
    w = c * NS + s
    pltpu.sync_copy(z1_r.at[pl.ds(640 * s, 640)], acc.at[pl.ds(640 * s, 640)])
    pltpu.sync_copy(dst_r.at[w], dstv)
    pltpu.sync_copy(ones_r, onesv)
    plsc.subcore_barrier()

    def body(j, carry):
        pltpu.sync_copy(onesv, acc.at[dstv.at[j]], add=True)
        return carry

    lax.fori_loop(0, _NCA, body, 0)
    plsc.subcore_barrier()
    pltpu.sync_copy(acc.at[pl.ds(640 * s, 640)], out_r.at[c, pl.ds(640 * s, 640)])


_GRP = 10
_CPG = 25


def _build_agg(edge_split: bool):
    ept = E // (NC * NS) if edge_split else E // NS
    k = 40 if edge_split else 80
    epg = ept // _GRP
    assert epg == _CPG * k
    rpt = _NPAD // NS
    mesh = plsc.VectorSubcoreMesh(core_axis_name="c", subcore_axis_name="s", num_cores=NC, num_subcores=NS)
    nw = NC * NS if edge_split else NS

    @functools.partial(
        pl.kernel,
        out_type=jax.ShapeDtypeStruct((NC, _NPAD, DIN), F32),
        mesh=mesh,
        scratch_types=[
            pltpu.VMEM((epg,), I32),
            pltpu.VMEM((_CPG, k), I32),
            pltpu.VMEM((k, DIN), F32),
            pltpu.VMEM_SHARED((_NPAD, DIN), F32),
            pltpu.SemaphoreType.DMA,
        ],
    )
    def agg(tab_r, src_r, dst_r, z2_r, out_r, sv, dv, rows, acc, sem):
        c = lax.axis_index("c")
        s = lax.axis_index("s")
        w = c * NS + s if edge_split else s
        pltpu.sync_copy(z2_r.at[pl.ds(rpt * s, rpt)], acc.at[pl.ds(rpt * s, rpt)])
        plsc.subcore_barrier()
        off = c * N

        def group(g, carry):
            pltpu.sync_copy(src_r.at[w, g], sv)
            pltpu.sync_copy(dst_r.at[w, g], dv)
            if not edge_split:
                def addoff(i, cy):
                    v = sv[pl.ds(i * 16, 16)]
                    sv[pl.ds(i * 16, 16)] = v + off
                    return cy

                lax.fori_loop(0, epg // 16, addoff, 0)

            def body(j, cy):
                pltpu.sync_copy(tab_r.at[sv.at[pl.ds(j * k, k)]], rows)
                pltpu.sync_copy(rows, acc.at[dv.at[j]], add=True)
                return cy

            lax.fori_loop(0, _CPG, body, 0)
            return carry

        lax.fori_loop(0, _GRP, group, 0)
        plsc.subcore_barrier()
        pltpu.sync_copy(acc.at[pl.ds(rpt * s, rpt)], out_r.at[c, pl.ds(rpt * s, rpt)])

    return agg


_agg_edge = _build_agg(True)
_agg_col = _build_agg(False)

_PREC = lax.Precision.HIGHEST


def _mm(a, b):
    return jnp.dot(a, b, precision=_PREC, preferred_element_type=F32)


def _b_body(degp, x, c2d_o, u0_o):
    dp = degp[...]
    deg = dp[0] + dp[1] + 1.0
    c = lax.rsqrt(deg)
    c2d_o[...] = c
    u0_o[...] = x[...] * c


def _b_kernel(degp, x):
    return pl.pallas_call(
        _b_body,
        out_shape=[
            jax.ShapeDtypeStruct((N, 1), F32),
            jax.ShapeDtypeStruct((N, DIN), F32),
        ],
    )(degp, x)


_RB = 1000
_NG = N // _RB


def _d1_body(p, u0, c2, w1, b1, o):
    cc = c2[...]
    t = (p[0] + p[1] + u0[...]) * cc
    h = jnp.maximum(_mm(t, w1[...]) + b1[...], 0.0)
    u = h * cc
    o[...] = jnp.stack([u[:, :DIN], u[:, DIN:]])


def _d1(p, u0, c2, w1, b1):
    return pl.pallas_call(
        _d1_body,
        grid=(_NG,),
        in_specs=[
            pl.BlockSpec((NC, _RB, DIN), lambda i: (0, i, 0)),
            pl.BlockSpec((_RB, DIN), lambda i: (i, 0)),
            pl.BlockSpec((_RB, 1), lambda i: (i, 0)),
            pl.BlockSpec((DIN, HID), lambda i: (0, 0)),
            pl.BlockSpec((1, HID), lambda i: (0, 0)),
        ],
        out_specs=pl.BlockSpec((NC, _RB, DIN), lambda i: (0, i, 0)),
        out_shape=jax.ShapeDtypeStruct((NC, N, DIN), F32),
    )(p, u0, c2, w1, b1)


def _d2_body(a, up, c2, wl, wr, b, o):
    cc = c2[...]
    tl = (a[0] + up[0]) * cc
    tr = (a[1] + up[1]) * cc
    h = jnp.maximum(_mm(tl, wl[...]) + _mm(tr, wr[...]) + b[...], 0.0)
    u = h * cc
    o[...] = jnp.stack([u[:, :DIN], u[:, DIN:]])


def _d2(a, up, c2, wl, wr, b):
    return pl.pallas_call(
        _d2_body,
        grid=(_NG,),
        in_specs=[
            pl.BlockSpec((NC, _RB, DIN), lambda i: (0, i, 0)),
            pl.BlockSpec((NC, _RB, DIN), lambda i: (0, i, 0)),
            pl.BlockSpec((_RB, 1), lambda i: (i, 0)),
            pl.BlockSpec((DIN, HID), lambda i: (0, 0)),
            pl.BlockSpec((DIN, HID), lambda i: (0, 0)),
            pl.BlockSpec((1, HID), lambda i: (0, 0)),
        ],
        out_specs=pl.BlockSpec((NC, _RB, DIN), lambda i: (0, i, 0)),
        out_shape=jax.ShapeDtypeStruct((NC, N, DIN), F32),
    )(a, up, c2, wl, wr, b)


def _d3_body(a, up, c2, bt, wl, wr, b, wfc, bfc, wp, bp, o, pooled, cnt):
    i = pl.program_id(0)
    cc = c2[...]
    tl = (a[0] + up[0]) * cc
    tr = (a[1] + up[1]) * cc
    h3 = _mm(tl, wl[...]) + _mm(tr, wr[...]) + b[...]

    gids = lax.broadcasted_iota(I32, (1, G), 1)
    oh = (bt[...] == gids).astype(F32)
    dn = (((0,), (0,)), ((), ()))

    @pl.when(i == 0)
    def _init():
        pooled[...] = jnp.zeros((G, HID), F32)
        cnt[...] = jnp.zeros((G, 1), F32)

    pooled[...] += lax.dot_general(oh, h3, dn, precision=_PREC,
                                   preferred_element_type=F32)
    cnt[...] += lax.dot_general(oh, jnp.ones((_RB, 1), F32), dn,
                                precision=_PREC, preferred_element_type=F32)

    @pl.when(i == _NG - 1)
    def _head():
        pm = pooled[...] / jnp.maximum(cnt[...], 1.0)
        o1 = _mm(pm, wfc[...]) + bfc[...]
        o2 = _mm(o1, wp[...]) + bp[...]
        o[...] = jax.nn.sigmoid(o2)


def _d3(a, up, c2, bt, wl, wr, b, wfc, bfc, wp, bp):
    return pl.pallas_call(
        _d3_body,
        grid=(_NG,),
        in_specs=[
            pl.BlockSpec((NC, _RB, DIN), lambda i: (0, i, 0)),
            pl.BlockSpec((NC, _RB, DIN), lambda i: (0, i, 0)),
            pl.BlockSpec((_RB, 1), lambda i: (i, 0)),
            pl.BlockSpec((_RB, 1), lambda i: (i, 0)),
            pl.BlockSpec((DIN, HID), lambda i: (0, 0)),
            pl.BlockSpec((DIN, HID), lambda i: (0, 0)),
            pl.BlockSpec((1, HID), lambda i: (0, 0)),
            pl.BlockSpec((HID, G), lambda i: (0, 0)),
            pl.BlockSpec((1, G), lambda i: (0, 0)),
            pl.BlockSpec((G, 1), lambda i: (0, 0)),
            pl.BlockSpec((1, 1), lambda i: (0, 0)),
        ],
        out_specs=pl.BlockSpec((G, 1), lambda i: (0, 0)),
        out_shape=jax.ShapeDtypeStruct((G, 1), F32),
        scratch_shapes=[
            pltpu.VMEM((G, HID), F32),
            pltpu.VMEM((G, 1), F32),
        ],
    )(a, up, c2, bt, wl, wr, b, wfc, bfc, wp, bp)


def kernel(x, edge_index, batch, W_emb, b_emb, W1, b1, W2, b2, W3, b3,
           Wfc1, bfc1, Wp, bp):
    del W_emb, b_emb
    src = edge_index[0].astype(I32)
    dst = edge_index[1].astype(I32)
    src_e = src.reshape(NC * NS, _GRP, 1000)
    dst_e = dst.reshape(NC * NS, _GRP, _CPG, 40)
    src_c = src.reshape(NS, _GRP, 2000)
    dst_c = dst.reshape(NS, _GRP, _CPG, 80)
    dst_a = dst.reshape(NC * NS, _NCA, _KA)
    ones_a = jnp.ones((_KA, 1), F32)
    z1 = jnp.zeros((_NPAD, 1), F32)
    z2 = jnp.zeros((_NPAD, DIN), F32)

    degp = _deg_kernel(dst_a, ones_a, z1)[:, :N, :]
    c2d, u0 = _b_kernel(degp, x)

    p1 = _agg_edge(u0, src_e, dst_e, z2)[:, :N, :]
    un1 = _d1(p1, u0, c2d, W1, b1.reshape(1, HID))

    a2 = _agg_col(un1.reshape(2 * N, DIN), src_c, dst_c, z2)[:, :N, :]
    un2 = _d2(a2, un1, c2d, W2[:DIN], W2[DIN:], b2.reshape(1, HID))

    a3 = _agg_col(un2.reshape(2 * N, DIN), src_c, dst_c, z2)[:, :N, :]
    out = _d3(a3, un2, c2d, batch.astype(I32).reshape(N, 1),
              W3[:DIN], W3[DIN:], b3.reshape(1, HID),
              Wfc1, bfc1.reshape(1, G), Wp, bp.reshape(1, 1))
    return out


# speedup vs baseline: 19.1729x; 1.6259x over previous
import functools

import jax
import jax.numpy as jnp
from jax import lax
from jax.experimental import pallas as pl
from jax.experimental.pallas import tpu as pltpu
from jax.experimental.pallas import tpu_sc as plsc

N = 10000
E = 320000
DIN = 128
HID = 256
G = 64
NC = 2
NS = 16
F32 = jnp.float32
I32 = jnp.int32

_KA = 40
_NCA = (E // (NC * NS)) // _KA
_NPAD = 10240

_deg_mesh = plsc.VectorSubcoreMesh(core_axis_name="c", subcore_axis_name="s", num_cores=NC, num_subcores=NS)


@functools.partial(
    pl.kernel,
    out_type=jax.ShapeDtypeStruct((NC, _NPAD, 1), F32),
    mesh=_deg_mesh,
    scratch_types=[
        pltpu.VMEM((_NCA, _KA), I32),
        pltpu.VMEM((_KA, 1), F32),
        pltpu.VMEM_SHARED((_NPAD, 1), F32),
    ],
)
def _deg_kernel(dst_r, ones_r, z1_r, out_r, dstv, onesv, acc):
    c = lax.axis_index("c")
    s = lax.axis_index("s")
    w = c * NS + s
    pltpu.sync_copy(z1_r.at[pl.ds(640 * s, 640)], acc.at[pl.ds(640 * s, 640)])
    pltpu.sync_copy(dst_r.at[w], dstv)
    pltpu.sync_copy(ones_r, onesv)
    plsc.subcore_barrier()

    def body(j, carry):
        pltpu.sync_copy(onesv, acc.at[dstv.at[j]], add=True)
        return carry

    lax.fori_loop(0, _NCA, body, 0)
    plsc.subcore_barrier()
    pltpu.sync_copy(acc.at[pl.ds(640 * s, 640)], out_r.at[c, pl.ds(640 * s, 640)])


_CPG = 25
_K = 80
_EPG = _CPG * _K


def _build_agg(edge_split: bool):
    ept = E // (NC * NS) if edge_split else E // NS
    grp = ept // _EPG
    rpt = _NPAD // NS
    mesh = plsc.VectorSubcoreMesh(core_axis_name="c", subcore_axis_name="s", num_cores=NC, num_subcores=NS)

    @functools.partial(
        pl.kernel,
        out_type=jax.ShapeDtypeStruct((NC, _NPAD, DIN), F32),
        mesh=mesh,
        scratch_types=[
            pltpu.VMEM((_EPG,), I32),
            pltpu.VMEM((_CPG, _K), I32),
            pltpu.VMEM((_K, DIN), F32),
            pltpu.VMEM((_K, DIN), F32),
            pltpu.VMEM_SHARED((_NPAD, DIN), F32),
            pltpu.SemaphoreType.DMA,
            pltpu.SemaphoreType.DMA,
        ],
    )
    def agg(tab_r, src_r, dst_r, z2_r, out_r, sv, dv, r0, r1, acc, s0, s1):
        c = lax.axis_index("c")
        s = lax.axis_index("s")
        w = c * NS + s if edge_split else s
        pltpu.sync_copy(z2_r.at[pl.ds(rpt * s, rpt)], acc.at[pl.ds(rpt * s, rpt)])
        plsc.subcore_barrier()
        off = c * N

        def gath(j):
            return tab_r.at[sv.at[pl.ds(j * _K, _K)]]

        def step(j, bufp, semp, bufq, semq):
            @pl.when(j + 1 < _CPG)
            def _pf():
                pltpu.async_copy(gath(j + 1), bufq, semq)

            pltpu.make_async_copy(gath(j), bufp, semp).wait()
            pltpu.sync_copy(bufp, acc.at[dv.at[j]], add=True)

        def group(g, carry):
            pltpu.sync_copy(src_r.at[w, g], sv)
            pltpu.sync_copy(dst_r.at[w, g], dv)
            if not edge_split:
                def addoff(i, cy):
                    v = sv[pl.ds(i * 16, 16)]
                    sv[pl.ds(i * 16, 16)] = v + off
                    return cy

                lax.fori_loop(0, _EPG // 16, addoff, 0)
            pltpu.async_copy(gath(0), r0, s0)

            def body(j, cy):
                @pl.when(j % 2 == 0)
                def _even():
                    step(j, r0, s0, r1, s1)

                @pl.when(j % 2 == 1)
                def _odd():
                    step(j, r1, s1, r0, s0)

                return cy

            lax.fori_loop(0, _CPG, body, 0)
            return carry

        lax.fori_loop(0, grp, group, 0)
        plsc.subcore_barrier()
        pltpu.sync_copy(acc.at[pl.ds(rpt * s, rpt)], out_r.at[c, pl.ds(rpt * s, rpt)])

    return agg


_agg_edge = _build_agg(True)
_agg_col = _build_agg(False)

_PREC = lax.Precision.HIGHEST


def _mm(a, b):
    return jnp.dot(a, b, precision=_PREC, preferred_element_type=F32)


def _b_body(degp, x, c2d_o, u0_o):
    dp = degp[...]
    deg = dp[0] + dp[1] + 1.0
    c = lax.rsqrt(deg)
    c2d_o[...] = c
    u0_o[...] = x[...] * c


def _b_kernel(degp, x):
    return pl.pallas_call(
        _b_body,
        out_shape=[
            jax.ShapeDtypeStruct((N, 1), F32),
            jax.ShapeDtypeStruct((N, DIN), F32),
        ],
    )(degp, x)


_RB = 1000
_NG = N // _RB


def _d1_body(p, u0, c2, w1, b1, o):
    cc = c2[...]
    t = (p[0] + p[1] + u0[...]) * cc
    h = jnp.maximum(_mm(t, w1[...]) + b1[...], 0.0)
    u = h * cc
    o[...] = jnp.stack([u[:, :DIN], u[:, DIN:]])


def _d1(p, u0, c2, w1, b1):
    return pl.pallas_call(
        _d1_body,
        grid=(_NG,),
        in_specs=[
            pl.BlockSpec((NC, _RB, DIN), lambda i: (0, i, 0)),
            pl.BlockSpec((_RB, DIN), lambda i: (i, 0)),
            pl.BlockSpec((_RB, 1), lambda i: (i, 0)),
            pl.BlockSpec((DIN, HID), lambda i: (0, 0)),
            pl.BlockSpec((1, HID), lambda i: (0, 0)),
        ],
        out_specs=pl.BlockSpec((NC, _RB, DIN), lambda i: (0, i, 0)),
        out_shape=jax.ShapeDtypeStruct((NC, N, DIN), F32),
    )(p, u0, c2, w1, b1)


def _d2_body(a, up, c2, wl, wr, b, o):
    cc = c2[...]
    tl = (a[0] + up[0]) * cc
    tr = (a[1] + up[1]) * cc
    h = jnp.maximum(_mm(tl, wl[...]) + _mm(tr, wr[...]) + b[...], 0.0)
    u = h * cc
    o[...] = jnp.stack([u[:, :DIN], u[:, DIN:]])


def _d2(a, up, c2, wl, wr, b):
    return pl.pallas_call(
        _d2_body,
        grid=(_NG,),
        in_specs=[
            pl.BlockSpec((NC, _RB, DIN), lambda i: (0, i, 0)),
            pl.BlockSpec((NC, _RB, DIN), lambda i: (0, i, 0)),
            pl.BlockSpec((_RB, 1), lambda i: (i, 0)),
            pl.BlockSpec((DIN, HID), lambda i: (0, 0)),
            pl.BlockSpec((DIN, HID), lambda i: (0, 0)),
            pl.BlockSpec((1, HID), lambda i: (0, 0)),
        ],
        out_specs=pl.BlockSpec((NC, _RB, DIN), lambda i: (0, i, 0)),
        out_shape=jax.ShapeDtypeStruct((NC, N, DIN), F32),
    )(a, up, c2, wl, wr, b)


def _d3_body(a, up, c2, bt, wl, wr, b, wfc, bfc, wp, bp, o, pooled, cnt):
    i = pl.program_id(0)
    cc = c2[...]
    tl = (a[0] + up[0]) * cc
    tr = (a[1] + up[1]) * cc
    h3 = _mm(tl, wl[...]) + _mm(tr, wr[...]) + b[...]

    gids = lax.broadcasted_iota(I32, (1, G), 1)
    oh = (bt[...] == gids).astype(F32)
    dn = (((0,), (0,)), ((), ()))

    @pl.when(i == 0)
    def _init():
        pooled[...] = jnp.zeros((G, HID), F32)
        cnt[...] = jnp.zeros((G, 1), F32)

    pooled[...] += lax.dot_general(oh, h3, dn, precision=_PREC,
                                   preferred_element_type=F32)
    cnt[...] += lax.dot_general(oh, jnp.ones((_RB, 1), F32), dn,
                                precision=_PREC, preferred_element_type=F32)

    @pl.when(i == _NG - 1)
    def _head():
        pm = pooled[...] / jnp.maximum(cnt[...], 1.0)
        o1 = _mm(pm, wfc[...]) + bfc[...]
        o2 = _mm(o1, wp[...]) + bp[...]
        o[...] = jax.nn.sigmoid(o2)


def _d3(a, up, c2, bt, wl, wr, b, wfc, bfc, wp, bp):
    return pl.pallas_call(
        _d3_body,
        grid=(_NG,),
        in_specs=[
            pl.BlockSpec((NC, _RB, DIN), lambda i: (0, i, 0)),
            pl.BlockSpec((NC, _RB, DIN), lambda i: (0, i, 0)),
            pl.BlockSpec((_RB, 1), lambda i: (i, 0)),
            pl.BlockSpec((_RB, 1), lambda i: (i, 0)),
            pl.BlockSpec((DIN, HID), lambda i: (0, 0)),
            pl.BlockSpec((DIN, HID), lambda i: (0, 0)),
            pl.BlockSpec((1, HID), lambda i: (0, 0)),
            pl.BlockSpec((HID, G), lambda i: (0, 0)),
            pl.BlockSpec((1, G), lambda i: (0, 0)),
            pl.BlockSpec((G, 1), lambda i: (0, 0)),
            pl.BlockSpec((1, 1), lambda i: (0, 0)),
        ],
        out_specs=pl.BlockSpec((G, 1), lambda i: (0, 0)),
        out_shape=jax.ShapeDtypeStruct((G, 1), F32),
        scratch_shapes=[
            pltpu.VMEM((G, HID), F32),
            pltpu.VMEM((G, 1), F32),
        ],
    )(a, up, c2, bt, wl, wr, b, wfc, bfc, wp, bp)


def kernel(x, edge_index, batch, W_emb, b_emb, W1, b1, W2, b2, W3, b3,
           Wfc1, bfc1, Wp, bp):
    del W_emb, b_emb
    src = edge_index[0].astype(I32)
    dst = edge_index[1].astype(I32)
    src_e = src.reshape(NC * NS, 5, _EPG)
    dst_e = dst.reshape(NC * NS, 5, _CPG, _K)
    src_c = src.reshape(NS, 10, _EPG)
    dst_c = dst.reshape(NS, 10, _CPG, _K)
    dst_a = dst.reshape(NC * NS, _NCA, _KA)
    ones_a = jnp.ones((_KA, 1), F32)
    z1 = jnp.zeros((_NPAD, 1), F32)
    z2 = jnp.zeros((_NPAD, DIN), F32)

    degp = _deg_kernel(dst_a, ones_a, z1)[:, :N, :]
    c2d, u0 = _b_kernel(degp, x)

    p1 = _agg_edge(u0, src_e, dst_e, z2)[:, :N, :]
    un1 = _d1(p1, u0, c2d, W1, b1.reshape(1, HID))

    a2 = _agg_col(un1.reshape(2 * N, DIN), src_c, dst_c, z2)[:, :N, :]
    un2 = _d2(a2, un1, c2d, W2[:DIN], W2[DIN:], b2.reshape(1, HID))

    a3 = _agg_col(un2.reshape(2 * N, DIN), src_c, dst_c, z2)[:, :N, :]
    out = _d3(a3, un2, c2d, batch.astype(I32).reshape(N, 1),
              W3[:DIN], W3[DIN:], b3.reshape(1, HID),
              Wfc1, bfc1.reshape(1, G), Wp, bp.reshape(1, 1))
    return out
